# baseline (speedup 1.0000x reference)
import jax
import jax.numpy as jnp
from jax.experimental import pallas as pl

K = 1024
D = 256
NB = 512


def _vq_block(lat_ref, cb_ref, out_ref, loss_ref):
    lt = lat_ref[0]
    flat = lt.T
    cb = cb_ref[...]
    f2 = jnp.sum(flat * flat, axis=1, keepdims=True)
    cb2 = jnp.sum(cb * cb, axis=1)
    mm = jax.lax.dot_general(flat, cb, (((1,), (1,)), ((), ())),
                             preferred_element_type=jnp.float32)
    dist = (f2 + cb2) - 2.0 * mm
    m = jnp.min(dist, axis=1, keepdims=True)
    iota = jax.lax.broadcasted_iota(jnp.int32, dist.shape, 1)
    idx = jnp.min(jnp.where(dist == m, iota, K), axis=1)
    oh = (iota == idx[:, None]).astype(jnp.float32)
    q = jax.lax.dot_general(oh, cb, (((1,), (0,)), ((), ())),
                            preferred_element_type=jnp.float32)
    diff = q - flat
    out_ref[0] = (flat + diff).T
    loss_ref[...] = jnp.full((1, 1, 1, 128), jnp.sum(diff * diff), jnp.float32)


def kernel(latents, vq_weight, codebook):
    b, d, t, h, w = latents.shape
    thw = t * h * w
    lat3 = latents.reshape(b, d, thw)
    nj = thw // NB
    out3, lossp = pl.pallas_call(
        _vq_block,
        grid=(b, nj),
        in_specs=[pl.BlockSpec((1, D, NB), lambda i, j: (i, 0, j)),
                  pl.BlockSpec((K, D), lambda i, j: (0, 0))],
        out_specs=[pl.BlockSpec((1, D, NB), lambda i, j: (i, 0, j)),
                   pl.BlockSpec((1, 1, 1, 128), lambda i, j: (i, j, 0, 0))],
        out_shape=[jax.ShapeDtypeStruct((b, d, thw), jnp.float32),
                   jax.ShapeDtypeStruct((b, nj, 1, 128), jnp.float32)],
    )(lat3, codebook)
    s = jnp.sum(lossp[:, :, 0, 0])
    mean = s / (b * thw * d)
    vq_loss = mean * vq_weight + mean
    return out3.reshape(b, d, t, h, w), vq_loss


# speedup vs baseline: 1.0654x; 1.0654x over previous
import jax
import jax.numpy as jnp
from jax.experimental import pallas as pl

K = 1024
D = 256
NB = 512


def _vq_block(lat_ref, cb_ref, out_ref, loss_ref):
    lt = lat_ref[0]
    cb = cb_ref[...]
    f2 = jnp.sum(lt * lt, axis=0)[:, None]
    cb2 = jnp.sum(cb * cb, axis=1)
    mm = jax.lax.dot_general(lt, cb, (((0,), (1,)), ((), ())),
                             preferred_element_type=jnp.float32)
    dist = (f2 + cb2) - 2.0 * mm
    m = jnp.min(dist, axis=1, keepdims=True)
    iota = jax.lax.broadcasted_iota(jnp.int32, dist.shape, 1)
    idx = jnp.min(jnp.where(dist == m, iota, K), axis=1)
    oh = (iota == idx[:, None]).astype(jnp.float32)
    qt = jax.lax.dot_general(cb, oh, (((0,), (1,)), ((), ())),
                             preferred_element_type=jnp.float32)
    dt = qt - lt
    out_ref[0] = lt + dt
    loss_ref[...] = jnp.full((1, 1, 1, 128), jnp.sum(dt * dt), jnp.float32)


def kernel(latents, vq_weight, codebook):
    b, d, t, h, w = latents.shape
    thw = t * h * w
    lat3 = latents.reshape(b, d, thw)
    nj = thw // NB
    out3, lossp = pl.pallas_call(
        _vq_block,
        grid=(b, nj),
        in_specs=[pl.BlockSpec((1, D, NB), lambda i, j: (i, 0, j)),
                  pl.BlockSpec((K, D), lambda i, j: (0, 0))],
        out_specs=[pl.BlockSpec((1, D, NB), lambda i, j: (i, 0, j)),
                   pl.BlockSpec((1, 1, 1, 128), lambda i, j: (i, j, 0, 0))],
        out_shape=[jax.ShapeDtypeStruct((b, d, thw), jnp.float32),
                   jax.ShapeDtypeStruct((b, nj, 1, 128), jnp.float32)],
    )(lat3, codebook)
    s = jnp.sum(lossp[:, :, 0, 0])
    mean = s / (b * thw * d)
    vq_loss = mean * vq_weight + mean
    return out3.reshape(b, d, t, h, w), vq_loss


# speedup vs baseline: 1.7318x; 1.6255x over previous
import jax
import jax.numpy as jnp
from jax.experimental import pallas as pl

K = 1024
D = 256
NB = 1024


def _vq_block(flat_ref, cb_ref, out_ref, loss_ref):
    flat = flat_ref[...]
    cb = cb_ref[...]
    f2 = jnp.sum(flat * flat, axis=1, keepdims=True)
    cb2 = jnp.sum(cb * cb, axis=1)
    mm = jax.lax.dot_general(flat, cb, (((1,), (1,)), ((), ())),
                             preferred_element_type=jnp.float32)
    dist = (f2 + cb2) - 2.0 * mm
    idx = jnp.argmin(dist, axis=1)
    iota = jax.lax.broadcasted_iota(jnp.int32, dist.shape, 1)
    oh = (iota == idx[:, None]).astype(jnp.float32)
    q = jax.lax.dot_general(oh, cb, (((1,), (0,)), ((), ())),
                            preferred_element_type=jnp.float32)
    diff = q - flat
    out_ref[...] = flat + diff
    loss_ref[...] = jnp.full((1, 1, 128), jnp.sum(diff * diff), jnp.float32)


def kernel(latents, vq_weight, codebook):
    lat = jnp.transpose(latents, (0, 2, 3, 4, 1))
    lat_shape = lat.shape
    flat = lat.reshape(-1, D)
    n = flat.shape[0]
    nblk = n // NB
    out, lossp = pl.pallas_call(
        _vq_block,
        grid=(nblk,),
        in_specs=[pl.BlockSpec((NB, D), lambda i: (i, 0)),
                  pl.BlockSpec((K, D), lambda i: (0, 0))],
        out_specs=[pl.BlockSpec((NB, D), lambda i: (i, 0)),
                   pl.BlockSpec((1, 1, 128), lambda i: (i, 0, 0))],
        out_shape=[jax.ShapeDtypeStruct((n, D), jnp.float32),
                   jax.ShapeDtypeStruct((nblk, 1, 128), jnp.float32)],
    )(flat, codebook)
    s = jnp.sum(lossp[:, 0, 0])
    mean = s / (n * D)
    vq_loss = mean * vq_weight + mean
    out5 = out.reshape(lat_shape)
    return jnp.transpose(out5, (0, 4, 1, 2, 3)), vq_loss
